# baseline (speedup 1.0000x reference)
import functools

import jax
import jax.numpy as jnp
from jax.experimental import pallas as pl
from jax.experimental.pallas import tpu as pltpu

_NEG_BIG = -1e30


def _outer(u, v):
    return jax.lax.dot_general(u, v, (((1,), (1,)), ((), ())),
                               preferred_element_type=jnp.float32)


def _fused_kernel(a_ref, x_ref, m_ref,
                  w1_ref, b1_ref, p1_ref, wp1_ref,
                  w2_ref, b2_ref, p2_ref, wp2_ref,
                  w3_ref, b3_ref, p3_ref, wp3_ref,
                  wa_ref, ba_ref, wb_ref, bb_ref,
                  o_ref, *, n):
    idx_r = jax.lax.broadcasted_iota(jnp.int32, (n, n), 0)
    idx_c = jax.lax.broadcasted_iota(jnp.int32, (n, n), 1)
    ones_i8 = jnp.ones((n, 1), jnp.int8)
    tie = idx_r < idx_c
    npair = a_ref.shape[0]

    gs = range(npair)

    def each(f):
        return [f(g) for g in gs]

    def stage(a_raw, a_bf, x, m, w_ref, b_ref, p_ref, wp_ref, k_keep):
        deg = each(lambda g: m[g] * jnp.dot(
            a_bf[g], m[g].astype(jnp.int8),
            preferred_element_type=jnp.int32).astype(jnp.float32))
        dinv = each(lambda g: jnp.where(
            deg[g] > 0.0, jax.lax.rsqrt(jnp.maximum(deg[g], 1e-12)), 0.0))
        dinv_b = each(lambda g: dinv[g].astype(jnp.bfloat16)
                      .astype(jnp.float32))
        dinv_t = each(lambda g: jax.lax.transpose(dinv_b[g], (1, 0)))
        a_norm = each(lambda g: (a_raw[g] * dinv_b[g]) * dinv_t[g])

        xw = each(lambda g: jnp.dot(x[g], w_ref[...],
                                    preferred_element_type=jnp.float32))
        h = each(lambda g: jnp.dot(a_norm[g], xw[g],
                                   preferred_element_type=jnp.float32)
                 + b_ref[...])
        h = each(lambda g: jnp.maximum(h[g], 0.0) * m[g])

        s = each(lambda g: jnp.dot(h[g], p_ref[...],
                                   preferred_element_type=jnp.float32))
        s = each(lambda g: jnp.where(m[g] > 0.0, s[g], _NEG_BIG))

        s_cols = each(lambda g: jax.lax.transpose(
            s[g].astype(jnp.bfloat16).astype(jnp.float32), (1, 0)))
        beats = each(lambda g: jnp.logical_or(
            s[g] > s_cols[g], jnp.logical_and(s[g] == s_cols[g], tie)))
        wins = each(lambda g: jnp.dot(
            beats[g].astype(jnp.int8), ones_i8,
            preferred_element_type=jnp.int32).astype(jnp.float32))
        keep = each(lambda g: jnp.where((n - 1.0) - wins[g] < k_keep,
                                        m[g], 0.0))

        x_new = each(lambda g: jnp.dot(
            h[g] * jnp.tanh(s[g]) * keep[g], wp_ref[...],
            preferred_element_type=jnp.float32) * keep[g])

        cnt = each(lambda g: jnp.sum(keep[g], axis=0, keepdims=True))
        avg = each(lambda g: jnp.sum(x_new[g], axis=0, keepdims=True)
                   / jnp.maximum(cnt[g], 1.0))
        mx = each(lambda g: jnp.max(
            jnp.where(keep[g] > 0.0, x_new[g], _NEG_BIG),
            axis=0, keepdims=True))
        mx = each(lambda g: jnp.where(cnt[g] > 0.0, mx[g], 0.0))
        ro = each(lambda g: jnp.concatenate([avg[g], mx[g]], axis=1))
        return x_new, keep, ro

    a_raw = each(lambda g: a_ref[g])
    a_bf = each(lambda g: a_raw[g].astype(jnp.int8))
    x0 = each(lambda g: x_ref[g])
    m0 = each(lambda g: m_ref[g])
    x1, m1, r1 = stage(a_raw, a_bf, x0, m0,
                       w1_ref, b1_ref, p1_ref, wp1_ref, n // 2)
    x2, m2, r2 = stage(a_raw, a_bf, x1, m1,
                       w2_ref, b2_ref, p2_ref, wp2_ref, n // 4)
    _, _, r3 = stage(a_raw, a_bf, x2, m2,
                     w3_ref, b3_ref, p3_ref, wp3_ref, n // 8)

    out = each(lambda g: jnp.dot(r1[g], wa_ref[...],
                                 preferred_element_type=jnp.float32)
               + ba_ref[...]
               + jnp.dot(r2[g], wb_ref[...],
                         preferred_element_type=jnp.float32)
               + bb_ref[...]
               + r3[g])
    for g in gs:
        o_ref[g] = out[g]


def kernel(gconv1_w, gconv1_b, lupool1_p, lupool1_w,
           gconv2_w, gconv2_b, lupool2_p, lupool2_w,
           gconv3_w, gconv3_b, lupool3_p, lupool3_w,
           cat_a_w, cat_a_b, cat_b_w, cat_b_b,
           a_raw, x_feat, mask, pad_dmap):
    del pad_dmap
    B, N, Fin = x_feat.shape
    E = cat_a_w.shape[1]

    const = lambda shape: pl.BlockSpec(shape, lambda i: (0,) * len(shape))
    weights = [
        gconv1_w, gconv1_b.reshape(1, -1), lupool1_p, lupool1_w,
        gconv2_w, gconv2_b.reshape(1, -1), lupool2_p, lupool2_w,
        gconv3_w, gconv3_b.reshape(1, -1), lupool3_p, lupool3_w,
        cat_a_w, cat_a_b.reshape(1, -1), cat_b_w, cat_b_b.reshape(1, -1),
    ]

    G = 4 if B % 4 == 0 else (2 if B % 2 == 0 else 1)
    out = pl.pallas_call(
        functools.partial(_fused_kernel, n=N),
        out_shape=jax.ShapeDtypeStruct((B, 1, E), jnp.float32),
        grid=(B // G,),
        in_specs=[
            pl.BlockSpec((G, N, N), lambda i: (i, 0, 0)),
            pl.BlockSpec((G, N, Fin), lambda i: (i, 0, 0)),
            pl.BlockSpec((G, N, 1), lambda i: (i, 0, 0)),
        ] + [const(w.shape) for w in weights],
        out_specs=pl.BlockSpec((G, 1, E), lambda i: (i, 0, 0)),
        compiler_params=pltpu.CompilerParams(
            dimension_semantics=("parallel",)),
    )(a_raw, x_feat, mask, *weights)
    return out.reshape(B, E)


# speedup vs baseline: 1.0883x; 1.0883x over previous
import functools

import jax
import jax.numpy as jnp
from jax.experimental import pallas as pl
from jax.experimental.pallas import tpu as pltpu

_NEG_BIG = -1e30


def _outer(u, v):
    return jax.lax.dot_general(u, v, (((1,), (1,)), ((), ())),
                               preferred_element_type=jnp.float32)


def _fused_kernel(a_ref, x_ref, m_ref,
                  w1_ref, b1_ref, p1_ref, wp1_ref,
                  w2_ref, b2_ref, p2_ref, wp2_ref,
                  w3_ref, b3_ref, p3_ref, wp3_ref,
                  wa_ref, ba_ref, wb_ref, bb_ref,
                  o_ref, *, n):
    idx_r = jax.lax.broadcasted_iota(jnp.int32, (n, n), 0)
    idx_c = jax.lax.broadcasted_iota(jnp.int32, (n, n), 1)
    ones_bf = jnp.ones((n, 1), jnp.bfloat16)
    tie = idx_r < idx_c
    npair = a_ref.shape[0]

    gs = range(npair)

    def each(f):
        return [f(g) for g in gs]

    def stage(a_raw, a_bf, x, m, w_ref, b_ref, p_ref, wp_ref, k_keep):
        deg = each(lambda g: m[g] * jnp.dot(
            a_bf[g], m[g].astype(jnp.bfloat16),
            preferred_element_type=jnp.float32))
        dinv = each(lambda g: jnp.where(
            deg[g] > 0.0, jax.lax.rsqrt(jnp.maximum(deg[g], 1e-12)), 0.0))
        dinv_b = each(lambda g: dinv[g].astype(jnp.bfloat16)
                      .astype(jnp.float32))
        dinv_t = each(lambda g: jax.lax.transpose(dinv_b[g], (1, 0)))
        a_norm = each(lambda g: ((a_raw[g] * dinv_b[g]) * dinv_t[g])
                      .astype(jnp.bfloat16))

        xw = each(lambda g: jnp.dot(x[g].astype(jnp.bfloat16),
                                    w_ref[...].astype(jnp.bfloat16),
                                    preferred_element_type=jnp.float32))
        h = each(lambda g: jnp.dot(a_norm[g], xw[g].astype(jnp.bfloat16),
                                   preferred_element_type=jnp.float32)
                 + b_ref[...])
        h = each(lambda g: jnp.maximum(h[g], 0.0) * m[g])

        s = each(lambda g: jnp.dot(h[g].astype(jnp.bfloat16),
                                   p_ref[...].astype(jnp.bfloat16),
                                   preferred_element_type=jnp.float32))
        s = each(lambda g: jnp.where(m[g] > 0.0, s[g], _NEG_BIG))

        s_cols = each(lambda g: jax.lax.transpose(
            s[g].astype(jnp.bfloat16).astype(jnp.float32), (1, 0)))
        beats = each(lambda g: jnp.logical_or(
            s[g] > s_cols[g], jnp.logical_and(s[g] == s_cols[g], tie)))
        wins = each(lambda g: jnp.dot(
            beats[g].astype(jnp.bfloat16), ones_bf,
            preferred_element_type=jnp.float32))
        keep = each(lambda g: jnp.where((n - 1.0) - wins[g] < k_keep,
                                        m[g], 0.0))

        x_new = each(lambda g: jnp.dot(
            (h[g] * jnp.tanh(s[g]) * keep[g]).astype(jnp.bfloat16),
            wp_ref[...].astype(jnp.bfloat16),
            preferred_element_type=jnp.float32) * keep[g])

        cnt = each(lambda g: jnp.sum(keep[g], axis=0, keepdims=True))
        avg = each(lambda g: jnp.sum(x_new[g], axis=0, keepdims=True)
                   / jnp.maximum(cnt[g], 1.0))
        mx = each(lambda g: jnp.max(
            jnp.where(keep[g] > 0.0, x_new[g], _NEG_BIG),
            axis=0, keepdims=True))
        mx = each(lambda g: jnp.where(cnt[g] > 0.0, mx[g], 0.0))
        ro = each(lambda g: jnp.concatenate([avg[g], mx[g]], axis=1))
        return x_new, keep, ro

    a_raw = each(lambda g: a_ref[g])
    a_bf = each(lambda g: a_raw[g].astype(jnp.bfloat16))
    x0 = each(lambda g: x_ref[g])
    m0 = each(lambda g: m_ref[g])
    x1, m1, r1 = stage(a_raw, a_bf, x0, m0,
                       w1_ref, b1_ref, p1_ref, wp1_ref, n // 2)
    x2, m2, r2 = stage(a_raw, a_bf, x1, m1,
                       w2_ref, b2_ref, p2_ref, wp2_ref, n // 4)
    _, _, r3 = stage(a_raw, a_bf, x2, m2,
                     w3_ref, b3_ref, p3_ref, wp3_ref, n // 8)

    out = each(lambda g: jnp.dot(r1[g], wa_ref[...],
                                 preferred_element_type=jnp.float32)
               + ba_ref[...]
               + jnp.dot(r2[g], wb_ref[...],
                         preferred_element_type=jnp.float32)
               + bb_ref[...]
               + r3[g])
    for g in gs:
        o_ref[g] = out[g]


def kernel(gconv1_w, gconv1_b, lupool1_p, lupool1_w,
           gconv2_w, gconv2_b, lupool2_p, lupool2_w,
           gconv3_w, gconv3_b, lupool3_p, lupool3_w,
           cat_a_w, cat_a_b, cat_b_w, cat_b_b,
           a_raw, x_feat, mask, pad_dmap):
    del pad_dmap
    B, N, Fin = x_feat.shape
    E = cat_a_w.shape[1]

    const = lambda shape: pl.BlockSpec(shape, lambda i: (0,) * len(shape))
    weights = [
        gconv1_w, gconv1_b.reshape(1, -1), lupool1_p, lupool1_w,
        gconv2_w, gconv2_b.reshape(1, -1), lupool2_p, lupool2_w,
        gconv3_w, gconv3_b.reshape(1, -1), lupool3_p, lupool3_w,
        cat_a_w, cat_a_b.reshape(1, -1), cat_b_w, cat_b_b.reshape(1, -1),
    ]

    G = 4 if B % 4 == 0 else (2 if B % 2 == 0 else 1)
    out = pl.pallas_call(
        functools.partial(_fused_kernel, n=N),
        out_shape=jax.ShapeDtypeStruct((B, 1, E), jnp.float32),
        grid=(B // G,),
        in_specs=[
            pl.BlockSpec((G, N, N), lambda i: (i, 0, 0)),
            pl.BlockSpec((G, N, Fin), lambda i: (i, 0, 0)),
            pl.BlockSpec((G, N, 1), lambda i: (i, 0, 0)),
        ] + [const(w.shape) for w in weights],
        out_specs=pl.BlockSpec((G, 1, E), lambda i: (i, 0, 0)),
        compiler_params=pltpu.CompilerParams(
            dimension_semantics=("parallel",)),
    )(a_raw, x_feat, mask, *weights)
    return out.reshape(B, E)


# speedup vs baseline: 1.1117x; 1.0215x over previous
import functools

import jax
import jax.numpy as jnp
from jax.experimental import pallas as pl
from jax.experimental.pallas import tpu as pltpu

_NEG_BIG = -1e30


def _outer(u, v):
    return jax.lax.dot_general(u, v, (((1,), (1,)), ((), ())),
                               preferred_element_type=jnp.float32)


def _fused_kernel(a_ref, x_ref, m_ref,
                  w1_ref, b1_ref, p1_ref, wp1_ref,
                  w2_ref, b2_ref, p2_ref, wp2_ref,
                  w3_ref, b3_ref, p3_ref, wp3_ref,
                  wa_ref, ba_ref, wb_ref, bb_ref,
                  o_ref, *, n):
    idx_r = jax.lax.broadcasted_iota(jnp.int32, (n, n), 0)
    idx_c = jax.lax.broadcasted_iota(jnp.int32, (n, n), 1)
    ones_bf = jnp.ones((n, 1), jnp.bfloat16)
    tie = idx_r < idx_c
    npair = a_ref.shape[0]

    gs = range(npair)

    def each(f):
        return [f(g) for g in gs]

    def stage(a_bf, x, m, w_ref, b_ref, p_ref, wp_ref, k_keep):
        deg = each(lambda g: m[g] * jnp.dot(
            a_bf[g], m[g].astype(jnp.bfloat16),
            preferred_element_type=jnp.float32))
        dinv = each(lambda g: jnp.where(
            deg[g] > 0.0, jax.lax.rsqrt(jnp.maximum(deg[g], 1e-12)), 0.0))
        dinv_b = each(lambda g: dinv[g].astype(jnp.bfloat16))
        dinv_t = each(lambda g: jax.lax.transpose(dinv_b[g], (1, 0)))
        a_norm = each(lambda g: (a_bf[g] * dinv_b[g]) * dinv_t[g])

        xw = each(lambda g: jnp.dot(x[g].astype(jnp.bfloat16),
                                    w_ref[...].astype(jnp.bfloat16),
                                    preferred_element_type=jnp.float32))
        h = each(lambda g: jnp.dot(a_norm[g], xw[g].astype(jnp.bfloat16),
                                   preferred_element_type=jnp.float32)
                 + b_ref[...])
        h = each(lambda g: jnp.maximum(h[g], 0.0) * m[g])

        s = each(lambda g: jnp.dot(h[g].astype(jnp.bfloat16),
                                   p_ref[...].astype(jnp.bfloat16),
                                   preferred_element_type=jnp.float32))
        s = each(lambda g: jnp.where(m[g] > 0.0, s[g], _NEG_BIG))

        s_cols = each(lambda g: jax.lax.transpose(
            s[g].astype(jnp.bfloat16).astype(jnp.float32), (1, 0)))
        beats = each(lambda g: jnp.logical_or(
            s[g] > s_cols[g], jnp.logical_and(s[g] == s_cols[g], tie)))
        wins = each(lambda g: jnp.dot(
            beats[g].astype(jnp.bfloat16), ones_bf,
            preferred_element_type=jnp.float32))
        keep = each(lambda g: jnp.where((n - 1.0) - wins[g] < k_keep,
                                        m[g], 0.0))

        x_new = each(lambda g: jnp.dot(
            (h[g] * jnp.tanh(s[g]) * keep[g]).astype(jnp.bfloat16),
            wp_ref[...].astype(jnp.bfloat16),
            preferred_element_type=jnp.float32) * keep[g])

        cnt = each(lambda g: jnp.sum(keep[g], axis=0, keepdims=True))
        avg = each(lambda g: jnp.sum(x_new[g], axis=0, keepdims=True)
                   / jnp.maximum(cnt[g], 1.0))
        mx = each(lambda g: jnp.max(
            jnp.where(keep[g] > 0.0, x_new[g], _NEG_BIG),
            axis=0, keepdims=True))
        mx = each(lambda g: jnp.where(cnt[g] > 0.0, mx[g], 0.0))
        ro = each(lambda g: jnp.concatenate([avg[g], mx[g]], axis=1))
        return x_new, keep, ro

    a_raw = each(lambda g: a_ref[g])
    a_bf = each(lambda g: a_raw[g].astype(jnp.bfloat16))
    x0 = each(lambda g: x_ref[g])
    m0 = each(lambda g: m_ref[g])
    x1, m1, r1 = stage(a_bf, x0, m0,
                       w1_ref, b1_ref, p1_ref, wp1_ref, n // 2)
    x2, m2, r2 = stage(a_bf, x1, m1,
                       w2_ref, b2_ref, p2_ref, wp2_ref, n // 4)
    _, _, r3 = stage(a_bf, x2, m2,
                     w3_ref, b3_ref, p3_ref, wp3_ref, n // 8)

    out = each(lambda g: jnp.dot(r1[g], wa_ref[...],
                                 preferred_element_type=jnp.float32)
               + ba_ref[...]
               + jnp.dot(r2[g], wb_ref[...],
                         preferred_element_type=jnp.float32)
               + bb_ref[...]
               + r3[g])
    for g in gs:
        o_ref[g] = out[g]


def kernel(gconv1_w, gconv1_b, lupool1_p, lupool1_w,
           gconv2_w, gconv2_b, lupool2_p, lupool2_w,
           gconv3_w, gconv3_b, lupool3_p, lupool3_w,
           cat_a_w, cat_a_b, cat_b_w, cat_b_b,
           a_raw, x_feat, mask, pad_dmap):
    del pad_dmap
    B, N, Fin = x_feat.shape
    E = cat_a_w.shape[1]

    const = lambda shape: pl.BlockSpec(shape, lambda i: (0,) * len(shape))
    weights = [
        gconv1_w, gconv1_b.reshape(1, -1), lupool1_p, lupool1_w,
        gconv2_w, gconv2_b.reshape(1, -1), lupool2_p, lupool2_w,
        gconv3_w, gconv3_b.reshape(1, -1), lupool3_p, lupool3_w,
        cat_a_w, cat_a_b.reshape(1, -1), cat_b_w, cat_b_b.reshape(1, -1),
    ]

    G = 4 if B % 4 == 0 else (2 if B % 2 == 0 else 1)
    out = pl.pallas_call(
        functools.partial(_fused_kernel, n=N),
        out_shape=jax.ShapeDtypeStruct((B, 1, E), jnp.float32),
        grid=(B // G,),
        in_specs=[
            pl.BlockSpec((G, N, N), lambda i: (i, 0, 0)),
            pl.BlockSpec((G, N, Fin), lambda i: (i, 0, 0)),
            pl.BlockSpec((G, N, 1), lambda i: (i, 0, 0)),
        ] + [const(w.shape) for w in weights],
        out_specs=pl.BlockSpec((G, 1, E), lambda i: (i, 0, 0)),
        compiler_params=pltpu.CompilerParams(
            dimension_semantics=("parallel",)),
    )(a_raw, x_feat, mask, *weights)
    return out.reshape(B, E)


# speedup vs baseline: 1.1327x; 1.0189x over previous
import functools

import jax
import jax.numpy as jnp
from jax.experimental import pallas as pl
from jax.experimental.pallas import tpu as pltpu

_NEG_BIG = -1e30


def _outer(u, v):
    return jax.lax.dot_general(u, v, (((1,), (1,)), ((), ())),
                               preferred_element_type=jnp.float32)


def _fused_kernel(a_ref, x_ref, m_ref,
                  w1_ref, b1_ref, p1_ref, wp1_ref,
                  w2_ref, b2_ref, p2_ref, wp2_ref,
                  w3_ref, b3_ref, p3_ref, wp3_ref,
                  wa_ref, ba_ref, wb_ref, bb_ref,
                  o_ref, *, n):
    idx_r = jax.lax.broadcasted_iota(jnp.int32, (n, n), 0)
    idx_c = jax.lax.broadcasted_iota(jnp.int32, (n, n), 1)
    ones_bf = jnp.ones((n, 1), jnp.bfloat16)
    tie = idx_r < idx_c
    npair = a_ref.shape[0]

    gs = range(npair)

    def each(f):
        return [f(g) for g in gs]

    def stage(a_bf, x, m, w_ref, b_ref, p_ref, wp_ref, k_keep):
        deg = each(lambda g: m[g] * jnp.dot(
            a_bf[g], m[g].astype(jnp.bfloat16),
            preferred_element_type=jnp.float32))
        dinv = each(lambda g: jnp.where(
            deg[g] > 0.0, jax.lax.rsqrt(jnp.maximum(deg[g], 1e-12)), 0.0))
        dinv_b = each(lambda g: dinv[g].astype(jnp.bfloat16))
        dinv_t = each(lambda g: jax.lax.transpose(dinv_b[g], (1, 0)))
        a_norm = each(lambda g: (a_bf[g] * dinv_b[g]) * dinv_t[g])

        xw = each(lambda g: jnp.dot(x[g].astype(jnp.bfloat16),
                                    w_ref[...].astype(jnp.bfloat16),
                                    preferred_element_type=jnp.float32))
        h = each(lambda g: jnp.dot(a_norm[g], xw[g].astype(jnp.bfloat16),
                                   preferred_element_type=jnp.float32)
                 + b_ref[...])
        h = each(lambda g: jnp.maximum(h[g], 0.0) * m[g])

        s = each(lambda g: jnp.dot(h[g].astype(jnp.bfloat16),
                                   p_ref[...].astype(jnp.bfloat16),
                                   preferred_element_type=jnp.float32))
        s = each(lambda g: jnp.where(m[g] > 0.0, s[g], _NEG_BIG))

        s_cols = each(lambda g: jax.lax.transpose(
            s[g].astype(jnp.bfloat16).astype(jnp.float32), (1, 0)))
        beats = each(lambda g: jnp.logical_or(
            s[g] > s_cols[g], jnp.logical_and(s[g] == s_cols[g], tie)))
        wins = each(lambda g: jnp.dot(
            beats[g].astype(jnp.bfloat16), ones_bf,
            preferred_element_type=jnp.float32))
        keep = each(lambda g: jnp.where((n - 1.0) - wins[g] < k_keep,
                                        m[g], 0.0))

        gate = each(lambda g: jnp.tanh(s[g]) * keep[g])
        x_new = each(lambda g: jnp.dot(
            (h[g] * gate[g]).astype(jnp.bfloat16),
            wp_ref[...].astype(jnp.bfloat16),
            preferred_element_type=jnp.float32) * keep[g])

        cnt = each(lambda g: jnp.sum(keep[g], axis=0, keepdims=True))
        avg = each(lambda g: jnp.sum(x_new[g], axis=0, keepdims=True)
                   / jnp.maximum(cnt[g], 1.0))
        mx = each(lambda g: jnp.max(
            jnp.where(keep[g] > 0.0, x_new[g], _NEG_BIG),
            axis=0, keepdims=True))
        mx = each(lambda g: jnp.where(cnt[g] > 0.0, mx[g], 0.0))
        ro = each(lambda g: jnp.concatenate([avg[g], mx[g]], axis=1))
        return x_new, keep, ro

    a_raw = each(lambda g: a_ref[g])
    a_bf = each(lambda g: a_raw[g].astype(jnp.bfloat16))
    x0 = each(lambda g: x_ref[g])
    m0 = each(lambda g: m_ref[g])
    x1, m1, r1 = stage(a_bf, x0, m0,
                       w1_ref, b1_ref, p1_ref, wp1_ref, n // 2)
    x2, m2, r2 = stage(a_bf, x1, m1,
                       w2_ref, b2_ref, p2_ref, wp2_ref, n // 4)
    _, _, r3 = stage(a_bf, x2, m2,
                     w3_ref, b3_ref, p3_ref, wp3_ref, n // 8)

    out = each(lambda g: jnp.dot(r1[g], wa_ref[...],
                                 preferred_element_type=jnp.float32)
               + ba_ref[...]
               + jnp.dot(r2[g], wb_ref[...],
                         preferred_element_type=jnp.float32)
               + bb_ref[...]
               + r3[g])
    for g in gs:
        o_ref[g] = out[g]


def kernel(gconv1_w, gconv1_b, lupool1_p, lupool1_w,
           gconv2_w, gconv2_b, lupool2_p, lupool2_w,
           gconv3_w, gconv3_b, lupool3_p, lupool3_w,
           cat_a_w, cat_a_b, cat_b_w, cat_b_b,
           a_raw, x_feat, mask, pad_dmap):
    del pad_dmap
    B, N, Fin = x_feat.shape
    E = cat_a_w.shape[1]

    const = lambda shape: pl.BlockSpec(shape, lambda i: (0,) * len(shape))
    weights = [
        gconv1_w, gconv1_b.reshape(1, -1), lupool1_p, lupool1_w,
        gconv2_w, gconv2_b.reshape(1, -1), lupool2_p, lupool2_w,
        gconv3_w, gconv3_b.reshape(1, -1), lupool3_p, lupool3_w,
        cat_a_w, cat_a_b.reshape(1, -1), cat_b_w, cat_b_b.reshape(1, -1),
    ]

    G = 4 if B % 4 == 0 else (2 if B % 2 == 0 else 1)
    out = pl.pallas_call(
        functools.partial(_fused_kernel, n=N),
        out_shape=jax.ShapeDtypeStruct((B, 1, E), jnp.float32),
        grid=(B // G,),
        in_specs=[
            pl.BlockSpec((G, N, N), lambda i: (i, 0, 0)),
            pl.BlockSpec((G, N, Fin), lambda i: (i, 0, 0)),
            pl.BlockSpec((G, N, 1), lambda i: (i, 0, 0)),
        ] + [const(w.shape) for w in weights],
        out_specs=pl.BlockSpec((G, 1, E), lambda i: (i, 0, 0)),
        compiler_params=pltpu.CompilerParams(
            dimension_semantics=("parallel",)),
    )(a_raw, x_feat, mask, *weights)
    return out.reshape(B, E)


# speedup vs baseline: 1.1814x; 1.0430x over previous
import functools

import jax
import jax.numpy as jnp
from jax.experimental import pallas as pl
from jax.experimental.pallas import tpu as pltpu

_NEG_BIG = -1e30


def _outer(u, v):
    return jax.lax.dot_general(u, v, (((1,), (1,)), ((), ())),
                               preferred_element_type=jnp.float32)


def _fused_kernel(a_ref, x_ref, m_ref,
                  w1_ref, b1_ref, p1_ref, wp1_ref,
                  w2_ref, b2_ref, p2_ref, wp2_ref,
                  w3_ref, b3_ref, p3_ref, wp3_ref,
                  wa_ref, ba_ref, wb_ref, bb_ref,
                  o_ref, *, n):
    idx_r = jax.lax.broadcasted_iota(jnp.int32, (n, n), 0)
    idx_c = jax.lax.broadcasted_iota(jnp.int32, (n, n), 1)
    ones_bf = jnp.ones((n, 1), jnp.bfloat16)
    tie = idx_r < idx_c
    npair = a_ref.shape[0]

    gs = range(npair)

    def each(f):
        return [f(g) for g in gs]

    def stage(a_bf, x, m, w_ref, b_ref, p_ref, wp_ref, k_keep):
        deg = each(lambda g: m[g] * jnp.dot(
            a_bf[g], m[g].astype(jnp.bfloat16),
            preferred_element_type=jnp.float32))
        dinv = each(lambda g: jnp.where(
            deg[g] > 0.0, jax.lax.rsqrt(jnp.maximum(deg[g], 1e-12)), 0.0))
        dinv_b = each(lambda g: dinv[g].astype(jnp.bfloat16))
        dinv_t = each(lambda g: jax.lax.transpose(dinv_b[g], (1, 0)))
        a_norm = each(lambda g: (a_bf[g] * dinv_b[g]) * dinv_t[g])

        xw = each(lambda g: jnp.dot(x[g].astype(jnp.bfloat16),
                                    w_ref[...].astype(jnp.bfloat16),
                                    preferred_element_type=jnp.float32))
        h = each(lambda g: jnp.dot(a_norm[g], xw[g].astype(jnp.bfloat16),
                                   preferred_element_type=jnp.float32)
                 + b_ref[...])
        h = each(lambda g: jnp.maximum(h[g], 0.0))

        s = each(lambda g: jnp.dot(h[g].astype(jnp.bfloat16),
                                   p_ref[...].astype(jnp.bfloat16),
                                   preferred_element_type=jnp.float32))
        s = each(lambda g: jnp.where(m[g] > 0.0, s[g], _NEG_BIG))

        s_cols = each(lambda g: jax.lax.transpose(
            s[g].astype(jnp.bfloat16).astype(jnp.float32), (1, 0)))
        beats = each(lambda g: jnp.logical_or(
            s[g] > s_cols[g], jnp.logical_and(s[g] == s_cols[g], tie)))
        wins = each(lambda g: jnp.dot(
            beats[g].astype(jnp.bfloat16), ones_bf,
            preferred_element_type=jnp.float32))
        keep = each(lambda g: jnp.where((n - 1.0) - wins[g] < k_keep,
                                        m[g], 0.0))

        gate = each(lambda g: jnp.tanh(s[g]) * keep[g])
        x_new = each(lambda g: jnp.dot(
            (h[g] * gate[g]).astype(jnp.bfloat16),
            wp_ref[...].astype(jnp.bfloat16),
            preferred_element_type=jnp.float32))

        cnt = each(lambda g: jnp.sum(keep[g], axis=0, keepdims=True))
        avg = each(lambda g: jnp.sum(x_new[g], axis=0, keepdims=True)
                   / jnp.maximum(cnt[g], 1.0))
        mx = each(lambda g: jnp.max(
            jnp.where(keep[g] > 0.0, x_new[g], _NEG_BIG),
            axis=0, keepdims=True))
        mx = each(lambda g: jnp.where(cnt[g] > 0.0, mx[g], 0.0))
        ro = each(lambda g: jnp.concatenate([avg[g], mx[g]], axis=1))
        return x_new, keep, ro

    a_raw = each(lambda g: a_ref[g])
    a_bf = each(lambda g: a_raw[g].astype(jnp.bfloat16))
    x0 = each(lambda g: x_ref[g])
    m0 = each(lambda g: m_ref[g])
    x1, m1, r1 = stage(a_bf, x0, m0,
                       w1_ref, b1_ref, p1_ref, wp1_ref, n // 2)
    x2, m2, r2 = stage(a_bf, x1, m1,
                       w2_ref, b2_ref, p2_ref, wp2_ref, n // 4)
    _, _, r3 = stage(a_bf, x2, m2,
                     w3_ref, b3_ref, p3_ref, wp3_ref, n // 8)

    out = each(lambda g: jnp.dot(r1[g], wa_ref[...],
                                 preferred_element_type=jnp.float32)
               + ba_ref[...]
               + jnp.dot(r2[g], wb_ref[...],
                         preferred_element_type=jnp.float32)
               + bb_ref[...]
               + r3[g])
    for g in gs:
        o_ref[g] = out[g]


def kernel(gconv1_w, gconv1_b, lupool1_p, lupool1_w,
           gconv2_w, gconv2_b, lupool2_p, lupool2_w,
           gconv3_w, gconv3_b, lupool3_p, lupool3_w,
           cat_a_w, cat_a_b, cat_b_w, cat_b_b,
           a_raw, x_feat, mask, pad_dmap):
    del pad_dmap
    B, N, Fin = x_feat.shape
    E = cat_a_w.shape[1]

    const = lambda shape: pl.BlockSpec(shape, lambda i: (0,) * len(shape))
    weights = [
        gconv1_w, gconv1_b.reshape(1, -1), lupool1_p, lupool1_w,
        gconv2_w, gconv2_b.reshape(1, -1), lupool2_p, lupool2_w,
        gconv3_w, gconv3_b.reshape(1, -1), lupool3_p, lupool3_w,
        cat_a_w, cat_a_b.reshape(1, -1), cat_b_w, cat_b_b.reshape(1, -1),
    ]

    G = 4 if B % 4 == 0 else (2 if B % 2 == 0 else 1)
    out = pl.pallas_call(
        functools.partial(_fused_kernel, n=N),
        out_shape=jax.ShapeDtypeStruct((B, 1, E), jnp.float32),
        grid=(B // G,),
        in_specs=[
            pl.BlockSpec((G, N, N), lambda i: (i, 0, 0)),
            pl.BlockSpec((G, N, Fin), lambda i: (i, 0, 0)),
            pl.BlockSpec((G, N, 1), lambda i: (i, 0, 0)),
        ] + [const(w.shape) for w in weights],
        out_specs=pl.BlockSpec((G, 1, E), lambda i: (i, 0, 0)),
        compiler_params=pltpu.CompilerParams(
            dimension_semantics=("parallel",)),
    )(a_raw, x_feat, mask, *weights)
    return out.reshape(B, E)
